# baseline (speedup 1.0000x reference)
import functools

import jax
import jax.numpy as jnp
from jax import lax
from jax.experimental import pallas as pl
from jax.experimental.pallas import tpu as pltpu
from jax.experimental.pallas import tpu_sc as plsc

_N = 512
_P = 64
_E = 128


def _chunk16(pT_c, xsT_c):
    r = pT_c[:, None, :] - xsT_c[:, :, None]
    u = r[0:8] + r[8:16]
    u = u[0:4] + u[4:8]
    u = u[0:2] + u[2:4]
    return u[0] + u[1]


def _tree_sum_p(pT, xsT):
    s0 = _chunk16(pT[0:16], xsT[0:16])
    s1 = _chunk16(pT[16:32], xsT[16:32])
    s2 = _chunk16(pT[32:48], xsT[32:48])
    s3 = _chunk16(pT[48:64], xsT[48:64])
    return ((s0 + s1) + s2) + s3


def _argmin_body(x_ref, perm_ref, out_ref):
    xb = x_ref[...]
    bb, h, wb, p = xb.shape
    pT = perm_ref[...].T
    xmin = jnp.min(xb, axis=1, keepdims=True)
    xmax = jnp.max(xb, axis=1, keepdims=True)
    den = (xmax - xmin) + jnp.float32(1e-8)
    xs = (xb - xmin) / den
    for s in range(bb):
        t_tok = h * wb
        xs2 = xs[s].reshape(t_tok, p)
        xsT = xs2.T
        t = _tree_sum_p(pT, xsT)
        at = jnp.abs(t)
        m = jnp.min(at, axis=1, keepdims=True)
        ii = jax.lax.broadcasted_iota(jnp.int32, at.shape, 1)
        idx = jnp.min(jnp.where(at == m, ii, _N), axis=1)
        out_ref[s] = idx


def _gather_sc(emb, idx_pad, b_total):
    info = plsc.get_sparse_core_info()
    nw = info.num_cores * info.num_subcores
    b_per_w = b_total // nw
    mesh = plsc.VectorSubcoreMesh(core_axis_name="c", subcore_axis_name="s")

    @functools.partial(
        pl.kernel, mesh=mesh,
        out_type=jax.ShapeDtypeStruct((b_total, _E), jnp.float32),
        scratch_types=[
            pltpu.VMEM((b_per_w,), jnp.int32),
            pltpu.VMEM((b_per_w, _E), jnp.float32),
            pltpu.SemaphoreType.DMA,
        ],
    )
    def k(emb_hbm, idx_hbm, out_hbm, idx_v, rows_v, sem):
        wid = lax.axis_index("s") * info.num_cores + lax.axis_index("c")
        base = wid * b_per_w
        pltpu.sync_copy(idx_hbm.at[pl.ds(base, b_per_w)], idx_v)
        pltpu.async_copy(emb_hbm.at[idx_v], rows_v, sem).wait()
        pltpu.sync_copy(rows_v, out_hbm.at[pl.ds(base, b_per_w)])

    return k(emb, idx_pad)


def kernel(x, perm, emb):
    b, h, w, p = x.shape
    t_tok = h * w
    perm2 = perm.reshape(_N, _P)
    idx = pl.pallas_call(
        _argmin_body,
        out_shape=jax.ShapeDtypeStruct((b, t_tok), jnp.int32),
    )(x, perm2)
    n_tok = b * t_tok
    b_total = ((n_tok + 255) // 256) * 256
    idx_pad = jnp.pad(idx.reshape(n_tok), (0, b_total - n_tok))
    rows = _gather_sc(emb, idx_pad, b_total)
    return rows[:n_tok].reshape(b, h, w, _E)


# speedup vs baseline: 2.3192x; 2.3192x over previous
import jax
import jax.numpy as jnp
from jax.experimental import pallas as pl

_N = 512
_P = 64
_E = 128


def _chunk16(pT_c, xsT_c):
    r = pT_c[:, None, :] - xsT_c[:, :, None]
    u = r[0:8] + r[8:16]
    u = u[0:4] + u[4:8]
    u = u[0:2] + u[2:4]
    return u[0] + u[1]


def _tree_sum_p(pT, xsT):
    s0 = _chunk16(pT[0:16], xsT[0:16])
    s1 = _chunk16(pT[16:32], xsT[16:32])
    s2 = _chunk16(pT[32:48], xsT[32:48])
    s3 = _chunk16(pT[48:64], xsT[48:64])
    return ((s0 + s1) + s2) + s3


def _body(x_ref, perm_ref, emb_ref, out_ref):
    xb = x_ref[...]
    bb, h, wb, p = xb.shape
    pT = perm_ref[...].T
    emb = emb_ref[...]
    xmin = jnp.min(xb, axis=1, keepdims=True)
    xmax = jnp.max(xb, axis=1, keepdims=True)
    den = (xmax - xmin) + jnp.float32(1e-8)
    xs = (xb - xmin) / den
    for s in range(0, bb, 2):
        t_tok = 2 * h * wb
        xs2 = xs[s:s + 2].reshape(t_tok, p)
        xsT = xs2.T
        t = _tree_sum_p(pT, xsT)
        at = jnp.abs(t)
        m = jnp.min(at, axis=1, keepdims=True)
        ii = jax.lax.broadcasted_iota(jnp.int32, at.shape, 1)
        idx = jnp.min(jnp.where(at == m, ii, _N), axis=1)
        oh = (jax.lax.broadcasted_iota(jnp.int32, (t_tok, _N), 1)
              == idx[:, None]).astype(jnp.float32)
        ob = jax.lax.dot_general(oh, emb,
                                 (((1,), (0,)), ((), ())),
                                 preferred_element_type=jnp.float32,
                                 precision=jax.lax.Precision.HIGHEST)
        out_ref[s:s + 2] = ob.reshape(2, h, wb, _E)


def kernel(x, perm, emb):
    b, h, w, p = x.shape
    perm2 = perm.reshape(_N, _P)
    return pl.pallas_call(
        _body,
        out_shape=jax.ShapeDtypeStruct((b, h, w, _E), jnp.float32),
    )(x, perm2, emb)


# speedup vs baseline: 2.3912x; 1.0311x over previous
import jax
import jax.numpy as jnp
from jax.experimental import pallas as pl

_N = 512
_P = 64
_E = 128


def _chunk16(pT_c, xsT_c):
    r = pT_c[:, None, :] - xsT_c[:, :, None]
    u = r[0:8] + r[8:16]
    u = u[0:4] + u[4:8]
    u = u[0:2] + u[2:4]
    return u[0] + u[1]


def _tree_sum_p(pT, xsT):
    s0 = _chunk16(pT[0:16], xsT[0:16])
    s1 = _chunk16(pT[16:32], xsT[16:32])
    s2 = _chunk16(pT[32:48], xsT[32:48])
    s3 = _chunk16(pT[48:64], xsT[48:64])
    return ((s0 + s1) + s2) + s3


def _body(x_ref, perm_ref, emb_ref, out_ref):
    xb = x_ref[...]
    bb, h, wb, p = xb.shape
    pT = perm_ref[...].T
    emb = emb_ref[...]
    xmin = jnp.min(xb, axis=1, keepdims=True)
    xmax = jnp.max(xb, axis=1, keepdims=True)
    den = (xmax - xmin) + jnp.float32(1e-8)
    xs = (xb - xmin) / den
    for s in range(bb):
        t_tok = h * wb
        xs2 = xs[s].reshape(t_tok, p)
        xsT = xs2.T
        t = _tree_sum_p(pT, xsT)
        at = jnp.abs(t)
        m = jnp.min(at, axis=1, keepdims=True)
        ii = jax.lax.broadcasted_iota(jnp.int32, at.shape, 1)
        idx = jnp.min(jnp.where(at == m, ii, _N), axis=1)
        oh = (jax.lax.broadcasted_iota(jnp.int32, (t_tok, _N), 1)
              == idx[:, None]).astype(jnp.float32)
        ob = jax.lax.dot_general(oh, emb,
                                 (((1,), (0,)), ((), ())),
                                 preferred_element_type=jnp.float32,
                                 precision=jax.lax.Precision.HIGHEST)
        out_ref[s] = ob.reshape(h, wb, _E)


def kernel(x, perm, emb):
    b, h, w, p = x.shape
    perm2 = perm.reshape(_N, _P)
    return pl.pallas_call(
        _body,
        out_shape=jax.ShapeDtypeStruct((b, h, w, _E), jnp.float32),
    )(x, perm2, emb)
